# Initial kernel scaffold; baseline (speedup 1.0000x reference)
import jax
import jax.numpy as jnp
from jax.experimental import pallas as pl


def kernel(x, edge_index, W_in, b_in, g_in, be_in, Wc, bc, gs, bs, W_att, b_att, W_out, b_out):
    raise NotImplementedError("write your pallas kernel here")


# speedup vs baseline: 2.5002x; 2.5002x over previous
import functools

import jax
import jax.numpy as jnp
from jax import lax
from jax.experimental import pallas as pl
from jax.experimental.pallas import tpu as pltpu
from jax.experimental.pallas import tpu_sc as plsc

N = 10000
E = 160000
D = 256
H = 256
C = 40
L = 6
EPS = 1e-5

HH = H // 2
NPAD = N + 8
NACC = 10112
NC = 2
NS = 16
NW = NC * NS
EPW = 10240
EPAD = EPW * NS
CHUNK = 128
ROWS_PER_SUB = NACC // NS


def _sc_agg_body(tbl0, tbl1, srcp, dstp, zeros_hbm, out, src_v, dst_v, rows_v,
                 acc_sp, gsem):
    cid = lax.axis_index("c")
    sid = lax.axis_index("s")
    base = sid * EPW

    def _run(tbl, out_half):
        r0 = sid * ROWS_PER_SUB
        pltpu.sync_copy(zeros_hbm.at[pl.ds(r0, ROWS_PER_SUB)],
                        acc_sp.at[pl.ds(r0, ROWS_PER_SUB)])
        plsc.subcore_barrier()

        def step(j, carry):
            off = base + j * CHUNK
            pltpu.sync_copy(srcp.at[pl.ds(off, CHUNK)], src_v)
            pltpu.sync_copy(dstp.at[pl.ds(off, CHUNK)], dst_v)
            pltpu.async_copy(tbl.at[src_v], rows_v, gsem).wait()
            pltpu.sync_copy(rows_v, acc_sp.at[dst_v], add=True)
            return carry

        lax.fori_loop(0, EPW // CHUNK, step, 0)
        plsc.subcore_barrier()
        pltpu.sync_copy(acc_sp.at[pl.ds(r0, ROWS_PER_SUB)],
                        out_half.at[pl.ds(r0, ROWS_PER_SUB)])

    @pl.when(cid == 0)
    def _():
        _run(tbl0, out.at[0])

    @pl.when(cid == 1)
    def _():
        _run(tbl1, out.at[1])


@jax.jit
def _sc_agg(tbl0, tbl1, srcp, dstp, zeros_hbm):
    mesh = plsc.VectorSubcoreMesh(core_axis_name="c", subcore_axis_name="s")
    return pl.kernel(
        _sc_agg_body,
        out_type=jax.ShapeDtypeStruct((2, NACC, HH), jnp.float32),
        mesh=mesh,
        scratch_types=[
            pltpu.VMEM((CHUNK,), jnp.int32),
            pltpu.VMEM((CHUNK,), jnp.int32),
            pltpu.VMEM((CHUNK, HH), jnp.float32),
            pltpu.VMEM_SHARED((NACC, HH), jnp.float32),
            pltpu.SemaphoreType.DMA,
        ],
    )(tbl0, tbl1, srcp, dstp, zeros_hbm)


def _bn(x, g, b):
    m = jnp.mean(x, axis=0, keepdims=True)
    d = x - m
    v = jnp.mean(d * d, axis=0, keepdims=True)
    return d * lax.rsqrt(v + EPS) * g + b


def _gate_relu(h, x_input, Wa_h, Wa_x, b_att):
    a = jax.nn.sigmoid(jnp.dot(h, Wa_h, preferred_element_type=jnp.float32)
                       + jnp.dot(x_input, Wa_x,
                                 preferred_element_type=jnp.float32) + b_att)
    return jnp.maximum((1.0 - a) * h + a * x_input, 0.0)


def _k_in_body(x, W_in, b_in, g_in, be_in, h0):
    h = jnp.dot(x[...], W_in[...], preferred_element_type=jnp.float32) + b_in[...]
    h0[...] = _bn(h, g_in[...], be_in[...])


def _k_fin_body(S, t0p, t1p, bc_p, gs_p, bs_p, degcol, hout):
    dinv = lax.rsqrt(degcol[...] + 1.0)
    Sfull = jnp.concatenate([S[0, :N] + t0p[:N], S[1, :N] + t1p[:N]], axis=1)
    agg = dinv * Sfull + bc_p[...]
    hout[...] = _bn(agg, gs_p[...], bs_p[...])


def _k_gm_body(h, x_input, Wa_h, Wa_x, b_att, Wc, degcol, t0, t1):
    dinv = lax.rsqrt(degcol[...] + 1.0)
    g = _gate_relu(h[...], x_input[...], Wa_h[...], Wa_x[...], b_att[...])
    hs = jnp.dot(g, Wc[...], preferred_element_type=jnp.float32) * dinv
    z = jnp.zeros((NPAD - N, HH), jnp.float32)
    t0[...] = jnp.concatenate([hs[:, :HH], z], axis=0)
    t1[...] = jnp.concatenate([hs[:, HH:], z], axis=0)


def _k_head_body(h, x_input, Wa_h, Wa_x, b_att, W_out, b_out, out):
    g = _gate_relu(h[...], x_input[...], Wa_h[...], Wa_x[...], b_att[...])
    o = jnp.dot(g, W_out[...], preferred_element_type=jnp.float32) + b_out[...]
    mx = jnp.max(o, axis=1, keepdims=True)
    ex = jnp.exp(o - mx)
    out[...] = o - mx - jnp.log(jnp.sum(ex, axis=1, keepdims=True))


_f32 = lambda *s: jax.ShapeDtypeStruct(s, jnp.float32)

_k_in = pl.pallas_call(_k_in_body, out_shape=_f32(N, H))
_k_fin = pl.pallas_call(_k_fin_body, out_shape=_f32(N, H))
_k_gm = pl.pallas_call(_k_gm_body, out_shape=(_f32(NPAD, HH), _f32(NPAD, HH)))
_k_head = pl.pallas_call(_k_head_body, out_shape=_f32(N, C))


def kernel(x, edge_index, W_in, b_in, g_in, be_in, Wc, bc, gs, bs, W_att,
           b_att, W_out, b_out):
    src = edge_index[0].astype(jnp.int32)
    dst = edge_index[1].astype(jnp.int32)
    pad = EPAD - E
    srcp = jnp.concatenate([src, jnp.full((pad,), N, jnp.int32)])
    dstp = jnp.concatenate([dst, jnp.zeros((pad,), jnp.int32)])
    zeros_hbm = jnp.zeros((NACC, HH), jnp.float32)
    ones_tbl = jnp.concatenate(
        [jnp.ones((N, HH), jnp.float32), jnp.zeros((NPAD - N, HH), jnp.float32)])

    degcol = _sc_agg(ones_tbl, ones_tbl, srcp, dstp, zeros_hbm)[0, :N, :1]

    Wa_h = W_att[:H]
    Wa_x = W_att[H:]

    h0 = _k_in(x, W_in, b_in, g_in, be_in)
    t0, t1 = _k_gm(h0, h0, Wa_h, Wa_x, b_att, Wc[0], degcol)
    for i in range(1, L + 1):
        S = _sc_agg(t0, t1, srcp, dstp, zeros_hbm)
        h = _k_fin(S, t0, t1, bc[i - 1], gs[i - 1], bs[i - 1], degcol)
        if i < L:
            t0, t1 = _k_gm(h, h0, Wa_h, Wa_x, b_att, Wc[i], degcol)
    return _k_head(h, h0, Wa_h, Wa_x, b_att, W_out, b_out)


# speedup vs baseline: 3.2410x; 1.2963x over previous
import functools

import jax
import jax.numpy as jnp
from jax import lax
from jax.experimental import pallas as pl
from jax.experimental.pallas import tpu as pltpu
from jax.experimental.pallas import tpu_sc as plsc

N = 10000
E = 160000
D = 256
H = 256
C = 40
L = 6
EPS = 1e-5

HH = H // 2
NPAD = N + 8
NACC = 10112
NC = 2
NS = 16
NW = NC * NS
EPW = 10240
EPAD = EPW * NS
CHUNK = 128
NCHUNK = EPW // CHUNK
HALFC = NCHUNK // 2
ROWS_PER_SUB = NACC // NS


def _sc_agg_body(tbl0, tbl1, src3, dst3, zeros_hbm, out, src_i, dst_i,
                 rows_a, rows_b, acc_sp, sem_a, sem_b):
    cid = lax.axis_index("c")
    sid = lax.axis_index("s")

    def _run(tbl, out_half):
        r0 = sid * ROWS_PER_SUB
        pltpu.sync_copy(zeros_hbm.at[pl.ds(r0, ROWS_PER_SUB)],
                        acc_sp.at[pl.ds(r0, ROWS_PER_SUB)])
        plsc.subcore_barrier()

        for p in range(2):
            pltpu.sync_copy(src3.at[sid, pl.ds(p * HALFC, HALFC)], src_i)
            pltpu.sync_copy(dst3.at[sid, pl.ds(p * HALFC, HALFC)], dst_i)
            pltpu.async_copy(tbl.at[src_i.at[0]], rows_a, sem_a)

            def step(k, carry):
                c0 = 2 * k
                pltpu.async_copy(tbl.at[src_i.at[c0 + 1]], rows_b, sem_b)
                pltpu.make_async_copy(tbl.at[src_i.at[0]], rows_a,
                                      sem_a).wait()
                pltpu.sync_copy(rows_a, acc_sp.at[dst_i.at[c0]], add=True)
                nxt = jnp.minimum(c0 + 2, HALFC - 1)
                pltpu.async_copy(tbl.at[src_i.at[nxt]], rows_a, sem_a)
                pltpu.make_async_copy(tbl.at[src_i.at[0]], rows_b,
                                      sem_b).wait()
                pltpu.sync_copy(rows_b, acc_sp.at[dst_i.at[c0 + 1]], add=True)
                return carry

            lax.fori_loop(0, HALFC // 2, step, 0)
            pltpu.make_async_copy(tbl.at[src_i.at[0]], rows_a, sem_a).wait()
        plsc.subcore_barrier()
        pltpu.sync_copy(acc_sp.at[pl.ds(r0, ROWS_PER_SUB)],
                        out_half.at[pl.ds(r0, ROWS_PER_SUB)])

    @pl.when(cid == 0)
    def _():
        _run(tbl0, out.at[0])

    @pl.when(cid == 1)
    def _():
        _run(tbl1, out.at[1])


@jax.jit
def _sc_agg(tbl0, tbl1, srcp, dstp, zeros_hbm):
    mesh = plsc.VectorSubcoreMesh(core_axis_name="c", subcore_axis_name="s")
    return pl.kernel(
        _sc_agg_body,
        out_type=jax.ShapeDtypeStruct((2, NACC, HH), jnp.float32),
        mesh=mesh,
        scratch_types=[
            pltpu.VMEM((HALFC, CHUNK), jnp.int32),
            pltpu.VMEM((HALFC, CHUNK), jnp.int32),
            pltpu.VMEM((CHUNK, HH), jnp.float32),
            pltpu.VMEM((CHUNK, HH), jnp.float32),
            pltpu.VMEM_SHARED((NACC, HH), jnp.float32),
            pltpu.SemaphoreType.DMA,
            pltpu.SemaphoreType.DMA,
        ],
    )(tbl0, tbl1, srcp, dstp, zeros_hbm)


def _bn(x, g, b):
    m = jnp.mean(x, axis=0, keepdims=True)
    d = x - m
    v = jnp.mean(d * d, axis=0, keepdims=True)
    return d * lax.rsqrt(v + EPS) * g + b


def _gate_relu(h, x_input, Wa_h, Wa_x, b_att):
    a = jax.nn.sigmoid(jnp.dot(h, Wa_h, preferred_element_type=jnp.float32)
                       + jnp.dot(x_input, Wa_x,
                                 preferred_element_type=jnp.float32) + b_att)
    return jnp.maximum((1.0 - a) * h + a * x_input, 0.0)


def _k_in_body(x, W_in, b_in, g_in, be_in, h0):
    h = jnp.dot(x[...], W_in[...], preferred_element_type=jnp.float32) + b_in[...]
    h0[...] = _bn(h, g_in[...], be_in[...])


def _k_fin_body(S, t0p, t1p, bc_p, gs_p, bs_p, degcol, hout):
    dinv = lax.rsqrt(degcol[...] + 1.0)
    Sfull = jnp.concatenate([S[0, :N] + t0p[:N], S[1, :N] + t1p[:N]], axis=1)
    agg = dinv * Sfull + bc_p[...]
    hout[...] = _bn(agg, gs_p[...], bs_p[...])


def _k_gm_body(h, x_input, Wa_h, Wa_x, b_att, Wc, degcol, t0, t1):
    dinv = lax.rsqrt(degcol[...] + 1.0)
    g = _gate_relu(h[...], x_input[...], Wa_h[...], Wa_x[...], b_att[...])
    hs = jnp.dot(g, Wc[...], preferred_element_type=jnp.float32) * dinv
    z = jnp.zeros((NPAD - N, HH), jnp.float32)
    t0[...] = jnp.concatenate([hs[:, :HH], z], axis=0)
    t1[...] = jnp.concatenate([hs[:, HH:], z], axis=0)


def _k_head_body(h, x_input, Wa_h, Wa_x, b_att, W_out, b_out, out):
    g = _gate_relu(h[...], x_input[...], Wa_h[...], Wa_x[...], b_att[...])
    o = jnp.dot(g, W_out[...], preferred_element_type=jnp.float32) + b_out[...]
    mx = jnp.max(o, axis=1, keepdims=True)
    ex = jnp.exp(o - mx)
    out[...] = o - mx - jnp.log(jnp.sum(ex, axis=1, keepdims=True))


_f32 = lambda *s: jax.ShapeDtypeStruct(s, jnp.float32)

_k_in = pl.pallas_call(_k_in_body, out_shape=_f32(N, H))
_k_fin = pl.pallas_call(_k_fin_body, out_shape=_f32(N, H))
_k_gm = pl.pallas_call(_k_gm_body, out_shape=(_f32(NPAD, HH), _f32(NPAD, HH)))
_k_head = pl.pallas_call(_k_head_body, out_shape=_f32(N, C))


def kernel(x, edge_index, W_in, b_in, g_in, be_in, Wc, bc, gs, bs, W_att,
           b_att, W_out, b_out):
    src = edge_index[0].astype(jnp.int32)
    dst = edge_index[1].astype(jnp.int32)
    pad = EPAD - E
    srcp = jnp.concatenate([src, jnp.full((pad,), N, jnp.int32)])
    dstp = jnp.concatenate([dst, jnp.zeros((pad,), jnp.int32)])
    srcp = srcp.reshape(NS, NCHUNK, CHUNK)
    dstp = dstp.reshape(NS, NCHUNK, CHUNK)
    zeros_hbm = jnp.zeros((NACC, HH), jnp.float32)
    ones_tbl = jnp.concatenate(
        [jnp.ones((N, HH), jnp.float32), jnp.zeros((NPAD - N, HH), jnp.float32)])

    degcol = _sc_agg(ones_tbl, ones_tbl, srcp, dstp, zeros_hbm)[0, :N, :1]

    Wa_h = W_att[:H]
    Wa_x = W_att[H:]

    h0 = _k_in(x, W_in, b_in, g_in, be_in)
    t0, t1 = _k_gm(h0, h0, Wa_h, Wa_x, b_att, Wc[0], degcol)
    for i in range(1, L + 1):
        S = _sc_agg(t0, t1, srcp, dstp, zeros_hbm)
        h = _k_fin(S, t0, t1, bc[i - 1], gs[i - 1], bs[i - 1], degcol)
        if i < L:
            t0, t1 = _k_gm(h, h0, Wa_h, Wa_x, b_att, Wc[i], degcol)
    return _k_head(h, h0, Wa_h, Wa_x, b_att, W_out, b_out)
